# baseline (speedup 1.0000x reference)
import functools

import jax
import jax.numpy as jnp
from jax import lax
from jax.experimental import pallas as pl
from jax.experimental.pallas import tpu as pltpu
from jax.experimental.pallas import tpu_sc as plsc

_DIM = 4096
_EXPERTS = 64
_TOKENS = 8192
_TILE = 1024
_HD = _DIM // 2

_NW = 32
_RPW = _TOKENS // _NW
_LANES = 16


def _softmax_kernel(xa_ref, xb_ref, wa_ref, wb_ref, b_ref, gs_ref, gst_ref):
    dn = (((1,), (1,)), ((), ()))
    la = jax.lax.dot_general(
        xa_ref[...], wa_ref[...], dn, preferred_element_type=jnp.float32
    )
    lb = jax.lax.dot_general(
        xb_ref[...], wb_ref[...], dn, preferred_element_type=jnp.float32
    )
    logits = la + lb + b_ref[...]
    m = jnp.max(logits, axis=1, keepdims=True)
    e = jnp.exp(logits - m)
    s = jnp.sum(e, axis=1, keepdims=True)
    gate = e / s
    gs_ref[...] = gate
    gst_ref[...] = gate.T


def _tc_softmax(x, W, b2):
    grid = (_TOKENS // _TILE,)
    return pl.pallas_call(
        _softmax_kernel,
        grid=grid,
        in_specs=[
            pl.BlockSpec((_TILE, _HD), lambda i: (i, 0)),
            pl.BlockSpec((_TILE, _HD), lambda i: (i, 1)),
            pl.BlockSpec((_EXPERTS, _HD), lambda i: (0, 0)),
            pl.BlockSpec((_EXPERTS, _HD), lambda i: (0, 1)),
            pl.BlockSpec((1, _EXPERTS), lambda i: (0, 0)),
        ],
        out_specs=[
            pl.BlockSpec((_TILE, _EXPERTS), lambda i: (i, 0)),
            pl.BlockSpec((_EXPERTS, _TILE), lambda i: (0, i)),
        ],
        out_shape=[
            jax.ShapeDtypeStruct((_TOKENS, _EXPERTS), jnp.float32),
            jax.ShapeDtypeStruct((_EXPERTS, _TOKENS), jnp.float32),
        ],
    )(x, x, W, W, b2)


def _sc_top2_kernel(gst_hbm, ts_hbm, ti_hbm, gv, tsv, tiv):
    wid = lax.axis_index("s") * 2 + lax.axis_index("c")
    base = wid * _RPW
    pltpu.sync_copy(gst_hbm.at[:, pl.ds(base, _RPW)], gv)

    iota = lax.iota(jnp.int32, _LANES)

    def block_body(bblk, _):
        col = bblk * _LANES
        neg = jnp.full((_LANES,), -jnp.inf, jnp.float32)
        zi = jnp.zeros((_LANES,), jnp.int32)

        def expert_body(e, carry):
            m1, i1, m2, i2 = carry
            v = gv[e, pl.ds(col, _LANES)]
            ev = jnp.full((_LANES,), 0, jnp.int32) + e
            gt1 = v > m1
            gt2 = v > m2
            m2n = jnp.where(gt1, m1, jnp.where(gt2, v, m2))
            i2n = jnp.where(gt1, i1, jnp.where(gt2, ev, i2))
            m1n = jnp.where(gt1, v, m1)
            i1n = jnp.where(gt1, ev, i1)
            return (m1n, i1n, m2n, i2n)

        m1, i1, m2, i2 = lax.fori_loop(
            0, _EXPERTS, expert_body, (neg, zi, neg, zi)
        )
        tsv[0, pl.ds(col, _LANES)] = m1
        tsv[1, pl.ds(col, _LANES)] = m2
        tiv[0, pl.ds(col, _LANES)] = i1
        tiv[1, pl.ds(col, _LANES)] = i2
        return _

    lax.fori_loop(0, _RPW // _LANES, block_body, None)
    pltpu.sync_copy(tsv, ts_hbm.at[:, pl.ds(base, _RPW)])
    pltpu.sync_copy(tiv, ti_hbm.at[:, pl.ds(base, _RPW)])


def _sc_top2(gst):
    mesh = plsc.VectorSubcoreMesh(core_axis_name="c", subcore_axis_name="s")
    fn = functools.partial(
        pl.kernel,
        mesh=mesh,
        out_type=[
            jax.ShapeDtypeStruct((2, _TOKENS), jnp.float32),
            jax.ShapeDtypeStruct((2, _TOKENS), jnp.int32),
        ],
        scratch_types=[
            pltpu.VMEM((_EXPERTS, _RPW), jnp.float32),
            pltpu.VMEM((2, _RPW), jnp.float32),
            pltpu.VMEM((2, _RPW), jnp.int32),
        ],
    )(_sc_top2_kernel)
    return fn(gst)


def kernel(x, W, b):
    b2 = b.reshape(1, _EXPERTS)
    gs, gst = _tc_softmax(x, W, b2)
    tst, tit = _sc_top2(gst)
    return (gs, tst.T, tit.T)


# speedup vs baseline: 1.2006x; 1.2006x over previous
import jax
import jax.numpy as jnp
from jax.experimental import pallas as pl

_DIM = 4096
_EXPERTS = 64
_TOKENS = 8192
_TILE = 1024
_HD = _DIM // 2


def _gate_kernel(xa_ref, xb_ref, wa_ref, wb_ref, b_ref, gs_ref, ts_ref, ti_ref):
    dn = (((1,), (1,)), ((), ()))
    la = jax.lax.dot_general(
        xa_ref[...], wa_ref[...], dn, preferred_element_type=jnp.float32
    )
    lb = jax.lax.dot_general(
        xb_ref[...], wb_ref[...], dn, preferred_element_type=jnp.float32
    )
    logits = la + lb + b_ref[...]
    m = jnp.max(logits, axis=1, keepdims=True)
    e = jnp.exp(logits - m)
    s = jnp.sum(e, axis=1, keepdims=True)
    gate = e / s
    gs_ref[...] = gate

    idx = jax.lax.broadcasted_iota(jnp.int32, gate.shape, 1)
    m1 = jnp.max(gate, axis=1, keepdims=True)
    i1 = jnp.min(jnp.where(gate == m1, idx, _EXPERTS), axis=1, keepdims=True)
    masked = jnp.where(idx == i1, -jnp.inf, gate)
    m2 = jnp.max(masked, axis=1, keepdims=True)
    i2 = jnp.min(jnp.where(masked == m2, idx, _EXPERTS), axis=1, keepdims=True)
    ts_ref[...] = jnp.concatenate([m1, m2], axis=1)
    ti_ref[...] = jnp.concatenate([i1, i2], axis=1)


def kernel(x, W, b):
    b2 = b.reshape(1, _EXPERTS)
    grid = (_TOKENS // _TILE,)
    out_shape = (
        jax.ShapeDtypeStruct((_TOKENS, _EXPERTS), jnp.float32),
        jax.ShapeDtypeStruct((_TOKENS, 2), jnp.float32),
        jax.ShapeDtypeStruct((_TOKENS, 2), jnp.int32),
    )
    gs, ts, ti = pl.pallas_call(
        _gate_kernel,
        grid=grid,
        in_specs=[
            pl.BlockSpec((_TILE, _HD), lambda i: (i, 0)),
            pl.BlockSpec((_TILE, _HD), lambda i: (i, 1)),
            pl.BlockSpec((_EXPERTS, _HD), lambda i: (0, 0)),
            pl.BlockSpec((_EXPERTS, _HD), lambda i: (0, 1)),
            pl.BlockSpec((1, _EXPERTS), lambda i: (0, 0)),
        ],
        out_specs=[
            pl.BlockSpec((_TILE, _EXPERTS), lambda i: (i, 0)),
            pl.BlockSpec((_TILE, 2), lambda i: (i, 0)),
            pl.BlockSpec((_TILE, 2), lambda i: (i, 0)),
        ],
        out_shape=out_shape,
    )(x, x, W, W, b2)
    return (gs, ts, ti)
